# baseline (speedup 1.0000x reference)
import functools

import jax
import jax.numpy as jnp
from jax import lax
from jax.experimental import pallas as pl
from jax.experimental.pallas import tpu as pltpu
from jax.experimental.pallas import tpu_sc as plsc

_NC = 2
_NS = 16


def _sc_gather(table, idx):
    n, d = table.shape
    e = idx.shape[0]
    nw = _NC * _NS
    per_w = e // nw
    dt = table.dtype
    chunk = 400 if d <= 64 else 200
    steps = per_w // chunk
    mesh = plsc.VectorSubcoreMesh(core_axis_name="c", subcore_axis_name="s")

    @functools.partial(
        pl.kernel,
        out_type=jax.ShapeDtypeStruct((e, d), dt),
        mesh=mesh,
        scratch_types=[
            pltpu.VMEM((per_w,), jnp.int32),
            [pltpu.VMEM((chunk, d), dt) for _ in range(4)],
            [pltpu.SemaphoreType.DMA for _ in range(4)],
            [pltpu.SemaphoreType.DMA for _ in range(4)],
        ],
    )
    def gather_kernel(table_hbm, idx_hbm, out_hbm, idx_f, rows, sg, ss):
        wid = lax.axis_index("s") * _NC + lax.axis_index("c")
        base = wid * per_w
        pltpu.sync_copy(idx_hbm.at[pl.ds(base, per_w)], idx_f)

        def issue_gather(c, b):
            pltpu.async_copy(
                table_hbm.at[idx_f.at[pl.ds(c * chunk, chunk)]], rows[b], sg[b])

        def wait_gather(b):
            pltpu.make_async_copy(
                out_hbm.at[pl.ds(0, chunk)], rows[b], sg[b]).wait()

        def issue_store(c, b):
            pltpu.async_copy(rows[b], out_hbm.at[pl.ds(base + c * chunk, chunk)],
                             ss[b])

        def wait_store(b):
            pltpu.make_async_copy(
                rows[b], out_hbm.at[pl.ds(0, chunk)], ss[b]).wait()

        def step(c, b, bp, prime=False):
            if not prime:
                wait_store(b)
            issue_gather(c, b)
            wait_gather(bp)
            issue_store(c - 2, bp)

        issue_gather(0, 0)
        issue_gather(1, 1)
        step(2, 2, 0, prime=True)
        step(3, 3, 1, prime=True)
        step(4, 0, 2)
        step(5, 1, 3)

        def body(g, carry):
            for j in range(4):
                c = 4 * g + 6 + j
                step(c, (2 + j) % 4, j)
            return carry

        nloop = (steps - 6) // 4
        lax.fori_loop(0, nloop, body, 0)
        for c in range(6 + 4 * nloop, steps):
            step(c, c % 4, (c - 2) % 4)
        for c in (steps - 2, steps - 1):
            b = c % 4
            wait_gather(b)
            issue_store(c, b)
        for c in range(steps - 4, steps):
            wait_store(c % 4)

    return gather_kernel(table, idx)


def _sc_scatter(msg, idx, zeros, n):
    e, d = msg.shape
    per_core = e // _NC
    per_w = per_core // _NS
    chunk = 80 if per_w % 80 == 0 else 40
    steps = per_w // chunk
    rpt = (-(-n // _NS) + 7) & ~7
    mesh = plsc.VectorSubcoreMesh(core_axis_name="c", subcore_axis_name="s")

    @functools.partial(
        pl.kernel,
        out_type=jax.ShapeDtypeStruct((_NC * n, d), jnp.float32),
        mesh=mesh,
        scratch_types=[
            [pltpu.VMEM((chunk,), jnp.int32) for _ in range(4)],
            [pltpu.VMEM((chunk, d), jnp.float32) for _ in range(4)],
            pltpu.VMEM_SHARED((n, d), jnp.float32),
            [pltpu.SemaphoreType.DMA for _ in range(4)],
            [pltpu.SemaphoreType.DMA for _ in range(4)],
            [pltpu.SemaphoreType.DMA for _ in range(4)],
        ],
    )
    def scatter_kernel(msg_hbm, idx_hbm, zeros_hbm, out_hbm,
                       idxb, msgb, acc, si, sm, sa):
        cc = lax.axis_index("c")
        s = lax.axis_index("s")
        rbase = pl.multiple_of(jnp.minimum(s * rpt, n - rpt), 8)
        pltpu.sync_copy(zeros_hbm, acc.at[pl.ds(rbase, rpt)])
        plsc.subcore_barrier()

        ebase = cc * per_core + s * per_w

        def issue_loads(c, b):
            off = ebase + c * chunk
            pltpu.async_copy(idx_hbm.at[pl.ds(off, chunk)], idxb[b], si[b])
            pltpu.async_copy(msg_hbm.at[pl.ds(off, chunk)], msgb[b], sm[b])

        def wait_loads(b):
            pltpu.make_async_copy(idx_hbm.at[pl.ds(0, chunk)], idxb[b],
                                  si[b]).wait()
            pltpu.make_async_copy(msg_hbm.at[pl.ds(0, chunk)], msgb[b],
                                  sm[b]).wait()

        def issue_add(b):
            pltpu.async_copy(msgb[b], acc.at[idxb[b]], sa[b], add=True)

        def wait_add(b):
            pltpu.make_async_copy(msgb[b], acc.at[pl.ds(0, chunk)],
                                  sa[b]).wait()

        def step(c, b, bp, prime=False):
            if not prime:
                wait_add(b)
            issue_loads(c, b)
            wait_loads(bp)
            issue_add(bp)

        issue_loads(0, 0)
        issue_loads(1, 1)
        step(2, 2, 0, prime=True)
        step(3, 3, 1, prime=True)
        step(4, 0, 2)
        step(5, 1, 3)

        def body(g, carry):
            for j in range(4):
                c = 4 * g + 6 + j
                step(c, (2 + j) % 4, j)
            return carry

        nloop = (steps - 6) // 4
        lax.fori_loop(0, nloop, body, 0)
        for c in range(6 + 4 * nloop, steps):
            step(c, c % 4, (c - 2) % 4)
        for c in (steps - 2, steps - 1):
            b = c % 4
            wait_loads(b)
            issue_add(b)
        for c in range(steps - 4, steps):
            wait_add(c % 4)

        plsc.subcore_barrier()
        pltpu.sync_copy(acc.at[pl.ds(rbase, rpt)],
                        out_hbm.at[pl.ds(cc * n + rbase, rpt)])

    return scatter_kernel(msg, idx, zeros)


def _dense_body(g_ref, e_ref, w_ref, b_ref, gg_ref, bb_ref, o_ref):
    d = e_ref.shape[1]
    g = g_ref[...].astype(jnp.bfloat16)
    ee = e_ref[...].astype(jnp.bfloat16)
    w = w_ref[...].astype(jnp.bfloat16)
    dn = (((1,), (1,)), ((), ()))
    c1 = lax.dot_general(g, w[:, :d], dn,
                         preferred_element_type=jnp.float32)
    c1 += lax.dot_general(ee, w[:, d:], dn,
                          preferred_element_type=jnp.float32)
    c1 += b_ref[...]
    mu = jnp.mean(c1, axis=-1, keepdims=True)
    var = jnp.mean((c1 - mu) ** 2, axis=-1, keepdims=True)
    c1 = (c1 - mu) / jnp.sqrt(var + 1e-5) * gg_ref[...] + bb_ref[...]
    o_ref[...] = jax.nn.sigmoid(c1[:, :d]) * jnp.tanh(c1[:, d:])


def _dense(gathered, edge, w1, b1, g_c1, be_c1):
    e, d = edge.shape
    be = 1600
    grid = e // be
    d2 = 2 * d
    return pl.pallas_call(
        _dense_body,
        grid=(grid,),
        in_specs=[
            pl.BlockSpec((be, d), lambda i: (i, 0)),
            pl.BlockSpec((be, d), lambda i: (i, 0)),
            pl.BlockSpec((d2, d2), lambda i: (0, 0)),
            pl.BlockSpec((1, d2), lambda i: (0, 0)),
            pl.BlockSpec((1, d2), lambda i: (0, 0)),
            pl.BlockSpec((1, d2), lambda i: (0, 0)),
        ],
        out_specs=pl.BlockSpec((be, d), lambda i: (i, 0)),
        out_shape=jax.ShapeDtypeStruct((e, d), jnp.float32),
    )(gathered, edge, w1, b1.reshape(1, d2), g_c1.reshape(1, d2),
      be_c1.reshape(1, d2))


def _final_body(n_ref, a0_ref, a1_ref, a2_ref, a3_ref, g_ref, b_ref, o_ref):
    agg = (a0_ref[...] + a1_ref[...]) + (a2_ref[...] + a3_ref[...])
    mu = jnp.mean(agg, axis=-1, keepdims=True)
    var = jnp.mean((agg - mu) ** 2, axis=-1, keepdims=True)
    ln = (agg - mu) / jnp.sqrt(var + 1e-5) * g_ref[...] + b_ref[...]
    o_ref[...] = jnp.tanh(n_ref[...] + ln)


def _final(node, aggs, g_bn, be_bn):
    n, d = node.shape
    bn = 1000
    blk = pl.BlockSpec((bn, d), lambda i: (i, 0))
    vec = pl.BlockSpec((1, d), lambda i: (0, 0))
    return pl.pallas_call(
        _final_body,
        grid=(n // bn,),
        in_specs=[blk] * 5 + [vec] * 2,
        out_specs=blk,
        out_shape=jax.ShapeDtypeStruct((n, d), jnp.float32),
    )(node, *aggs, g_bn.reshape(1, d), be_bn.reshape(1, d))


def kernel(node_embedding, edge_embedding, i, W1, b1, g_c1, be_c1, g_bn, be_bn):
    n, d = node_embedding.shape
    e = i.shape[0]
    idx = i.astype(jnp.int32)
    rpt = (-(-n // _NS) + 7) & ~7
    zeros = jnp.zeros((rpt, d), jnp.float32)
    h = e // 2
    idx0, idx1 = idx[:h], idx[h:]
    g0 = _sc_gather(node_embedding, idx0)
    g1 = _sc_gather(node_embedding, idx1)
    m0 = _dense(g0, edge_embedding[:h], W1, b1, g_c1, be_c1)
    m1 = _dense(g1, edge_embedding[h:], W1, b1, g_c1, be_c1)
    a0 = _sc_scatter(m0, idx0, zeros, n)
    a1 = _sc_scatter(m1, idx1, zeros, n)
    return _final(node_embedding, (a0[:n], a0[n:], a1[:n], a1[n:]),
                  g_bn, be_bn)


# speedup vs baseline: 1.0677x; 1.0677x over previous
import functools

import jax
import jax.numpy as jnp
from jax import lax
from jax.experimental import pallas as pl
from jax.experimental.pallas import tpu as pltpu
from jax.experimental.pallas import tpu_sc as plsc

_NC = 2
_NS = 16


def _sc_gather(table, idx):
    n, d = table.shape
    e = idx.shape[0]
    nw = _NC * _NS
    per_w = e // nw
    dt = table.dtype
    chunk = 400 if d <= 64 else 200
    steps = per_w // chunk
    mesh = plsc.VectorSubcoreMesh(core_axis_name="c", subcore_axis_name="s")

    @functools.partial(
        pl.kernel,
        out_type=jax.ShapeDtypeStruct((e, d), dt),
        mesh=mesh,
        scratch_types=[
            pltpu.VMEM((per_w,), jnp.int32),
            [pltpu.VMEM((chunk, d), dt) for _ in range(4)],
            [pltpu.SemaphoreType.DMA for _ in range(4)],
            [pltpu.SemaphoreType.DMA for _ in range(4)],
        ],
    )
    def gather_kernel(table_hbm, idx_hbm, out_hbm, idx_f, rows, sg, ss):
        wid = lax.axis_index("s") * _NC + lax.axis_index("c")
        base = wid * per_w
        pltpu.sync_copy(idx_hbm.at[pl.ds(base, per_w)], idx_f)

        def issue_gather(c, b):
            pltpu.async_copy(
                table_hbm.at[idx_f.at[pl.ds(c * chunk, chunk)]], rows[b], sg[b])

        def wait_gather(b):
            pltpu.make_async_copy(
                out_hbm.at[pl.ds(0, chunk)], rows[b], sg[b]).wait()

        def issue_store(c, b):
            pltpu.async_copy(rows[b], out_hbm.at[pl.ds(base + c * chunk, chunk)],
                             ss[b])

        def wait_store(b):
            pltpu.make_async_copy(
                rows[b], out_hbm.at[pl.ds(0, chunk)], ss[b]).wait()

        def step(c, b, bp, prime=False):
            if not prime:
                wait_store(b)
            issue_gather(c, b)
            wait_gather(bp)
            issue_store(c - 3, bp)

        issue_gather(0, 0)
        issue_gather(1, 1)
        issue_gather(2, 2)
        step(3, 3, 0, prime=True)

        def body(g, carry):
            for j in range(4):
                c = 4 * g + 4 + j
                step(c, j, (j + 1) % 4)
            return carry

        nloop = (steps - 4) // 4
        lax.fori_loop(0, nloop, body, 0)
        for c in range(4 + 4 * nloop, steps):
            step(c, c % 4, (c - 3) % 4)
        for c in (steps - 3, steps - 2, steps - 1):
            b = c % 4
            wait_gather(b)
            issue_store(c, b)
        for c in range(steps - 4, steps):
            wait_store(c % 4)

    return gather_kernel(table, idx)


def _sc_scatter(msg, idx, zeros, n):
    e, d = msg.shape
    per_core = e // _NC
    per_w = per_core // _NS
    chunk = 80 if per_w % 80 == 0 else 40
    steps = per_w // chunk
    rpt = (-(-n // _NS) + 7) & ~7
    mesh = plsc.VectorSubcoreMesh(core_axis_name="c", subcore_axis_name="s")

    @functools.partial(
        pl.kernel,
        out_type=jax.ShapeDtypeStruct((_NC * n, d), jnp.float32),
        mesh=mesh,
        scratch_types=[
            [pltpu.VMEM((chunk,), jnp.int32) for _ in range(4)],
            [pltpu.VMEM((chunk, d), jnp.float32) for _ in range(4)],
            pltpu.VMEM_SHARED((n, d), jnp.float32),
            [pltpu.SemaphoreType.DMA for _ in range(4)],
            [pltpu.SemaphoreType.DMA for _ in range(4)],
            [pltpu.SemaphoreType.DMA for _ in range(4)],
        ],
    )
    def scatter_kernel(msg_hbm, idx_hbm, zeros_hbm, out_hbm,
                       idxb, msgb, acc, si, sm, sa):
        cc = lax.axis_index("c")
        s = lax.axis_index("s")
        rbase = pl.multiple_of(jnp.minimum(s * rpt, n - rpt), 8)
        pltpu.sync_copy(zeros_hbm, acc.at[pl.ds(rbase, rpt)])
        plsc.subcore_barrier()

        ebase = cc * per_core + s * per_w

        def issue_loads(c, b):
            off = ebase + c * chunk
            pltpu.async_copy(idx_hbm.at[pl.ds(off, chunk)], idxb[b], si[b])
            pltpu.async_copy(msg_hbm.at[pl.ds(off, chunk)], msgb[b], sm[b])

        def wait_loads(b):
            pltpu.make_async_copy(idx_hbm.at[pl.ds(0, chunk)], idxb[b],
                                  si[b]).wait()
            pltpu.make_async_copy(msg_hbm.at[pl.ds(0, chunk)], msgb[b],
                                  sm[b]).wait()

        def issue_add(b):
            pltpu.async_copy(msgb[b], acc.at[idxb[b]], sa[b], add=True)

        def wait_add(b):
            pltpu.make_async_copy(msgb[b], acc.at[pl.ds(0, chunk)],
                                  sa[b]).wait()

        def step(c, b, bp, prime=False):
            if not prime:
                wait_add(b)
            issue_loads(c, b)
            wait_loads(bp)
            issue_add(bp)

        issue_loads(0, 0)
        issue_loads(1, 1)
        step(2, 2, 0, prime=True)
        step(3, 3, 1, prime=True)
        step(4, 0, 2)
        step(5, 1, 3)

        def body(g, carry):
            for j in range(4):
                c = 4 * g + 6 + j
                step(c, (2 + j) % 4, j)
            return carry

        nloop = (steps - 6) // 4
        lax.fori_loop(0, nloop, body, 0)
        for c in range(6 + 4 * nloop, steps):
            step(c, c % 4, (c - 2) % 4)
        for c in (steps - 2, steps - 1):
            b = c % 4
            wait_loads(b)
            issue_add(b)
        for c in range(steps - 4, steps):
            wait_add(c % 4)

        plsc.subcore_barrier()
        pltpu.sync_copy(acc.at[pl.ds(rbase, rpt)],
                        out_hbm.at[pl.ds(cc * n + rbase, rpt)])

    return scatter_kernel(msg, idx, zeros)


def _dense_body(g_ref, e_ref, w_ref, b_ref, gg_ref, bb_ref, o_ref):
    d = e_ref.shape[1]
    g = g_ref[...].astype(jnp.bfloat16)
    ee = e_ref[...].astype(jnp.bfloat16)
    w = w_ref[...].astype(jnp.bfloat16)
    dn = (((1,), (1,)), ((), ()))
    c1 = lax.dot_general(g, w[:, :d], dn,
                         preferred_element_type=jnp.float32)
    c1 += lax.dot_general(ee, w[:, d:], dn,
                          preferred_element_type=jnp.float32)
    c1 += b_ref[...]
    mu = jnp.mean(c1, axis=-1, keepdims=True)
    var = jnp.mean((c1 - mu) ** 2, axis=-1, keepdims=True)
    c1 = (c1 - mu) / jnp.sqrt(var + 1e-5) * gg_ref[...] + bb_ref[...]
    o_ref[...] = jax.nn.sigmoid(c1[:, :d]) * jnp.tanh(c1[:, d:])


def _dense(gathered, edge, w1, b1, g_c1, be_c1):
    e, d = edge.shape
    be = 1600
    grid = e // be
    d2 = 2 * d
    return pl.pallas_call(
        _dense_body,
        grid=(grid,),
        in_specs=[
            pl.BlockSpec((be, d), lambda i: (i, 0)),
            pl.BlockSpec((be, d), lambda i: (i, 0)),
            pl.BlockSpec((d2, d2), lambda i: (0, 0)),
            pl.BlockSpec((1, d2), lambda i: (0, 0)),
            pl.BlockSpec((1, d2), lambda i: (0, 0)),
            pl.BlockSpec((1, d2), lambda i: (0, 0)),
        ],
        out_specs=pl.BlockSpec((be, d), lambda i: (i, 0)),
        out_shape=jax.ShapeDtypeStruct((e, d), jnp.float32),
    )(gathered, edge, w1, b1.reshape(1, d2), g_c1.reshape(1, d2),
      be_c1.reshape(1, d2))


def _final_body(n_ref, a0_ref, a1_ref, g_ref, b_ref, o_ref):
    agg = a0_ref[...] + a1_ref[...]
    mu = jnp.mean(agg, axis=-1, keepdims=True)
    var = jnp.mean((agg - mu) ** 2, axis=-1, keepdims=True)
    ln = (agg - mu) / jnp.sqrt(var + 1e-5) * g_ref[...] + b_ref[...]
    o_ref[...] = jnp.tanh(n_ref[...] + ln)


def _final(node, aggs, g_bn, be_bn):
    n, d = node.shape
    bn = 1000
    blk = pl.BlockSpec((bn, d), lambda i: (i, 0))
    vec = pl.BlockSpec((1, d), lambda i: (0, 0))
    return pl.pallas_call(
        _final_body,
        grid=(n // bn,),
        in_specs=[blk] * 3 + [vec] * 2,
        out_specs=blk,
        out_shape=jax.ShapeDtypeStruct((n, d), jnp.float32),
    )(node, *aggs, g_bn.reshape(1, d), be_bn.reshape(1, d))


def kernel(node_embedding, edge_embedding, i, W1, b1, g_c1, be_c1, g_bn, be_bn):
    n, d = node_embedding.shape
    e = i.shape[0]
    idx = i.astype(jnp.int32)
    rpt = (-(-n // _NS) + 7) & ~7
    zeros = jnp.zeros((rpt, d), jnp.float32)
    gathered = _sc_gather(node_embedding, idx)
    msg = _dense(gathered, edge_embedding, W1, b1, g_c1, be_c1)
    agg2 = _sc_scatter(msg, idx, zeros, n)
    return _final(node_embedding, (agg2[:n], agg2[n:]), g_bn, be_bn)


# speedup vs baseline: 1.1822x; 1.1073x over previous
import functools

import jax
import jax.numpy as jnp
from jax import lax
from jax.experimental import pallas as pl
from jax.experimental.pallas import tpu as pltpu
from jax.experimental.pallas import tpu_sc as plsc

_NC = 2
_NS = 16


def _sc_gather(table, idx):
    n, d = table.shape
    e = idx.shape[0]
    nw = _NC * _NS
    per_w = e // nw
    dt = table.dtype
    chunk = 400 if d <= 64 else 200
    steps = per_w // chunk
    mesh = plsc.VectorSubcoreMesh(core_axis_name="c", subcore_axis_name="s")

    @functools.partial(
        pl.kernel,
        out_type=jax.ShapeDtypeStruct((e, d), dt),
        mesh=mesh,
        scratch_types=[
            pltpu.VMEM((per_w,), jnp.int32),
            [pltpu.VMEM((chunk, d), dt) for _ in range(4)],
            [pltpu.SemaphoreType.DMA for _ in range(4)],
            [pltpu.SemaphoreType.DMA for _ in range(4)],
        ],
    )
    def gather_kernel(table_hbm, idx_hbm, out_hbm, idx_f, rows, sg, ss):
        wid = lax.axis_index("s") * _NC + lax.axis_index("c")
        base = wid * per_w
        pltpu.sync_copy(idx_hbm.at[pl.ds(base, per_w)], idx_f)

        def issue_gather(c, b):
            pltpu.async_copy(
                table_hbm.at[idx_f.at[pl.ds(c * chunk, chunk)]], rows[b], sg[b])

        def wait_gather(b):
            pltpu.make_async_copy(
                out_hbm.at[pl.ds(0, chunk)], rows[b], sg[b]).wait()

        def issue_store(c, b):
            pltpu.async_copy(rows[b], out_hbm.at[pl.ds(base + c * chunk, chunk)],
                             ss[b])

        def wait_store(b):
            pltpu.make_async_copy(
                rows[b], out_hbm.at[pl.ds(0, chunk)], ss[b]).wait()

        def step(c, b, bp, prime=False):
            if not prime:
                wait_store(b)
            issue_gather(c, b)
            wait_gather(bp)
            issue_store(c - 3, bp)

        issue_gather(0, 0)
        issue_gather(1, 1)
        issue_gather(2, 2)
        step(3, 3, 0, prime=True)

        def body(g, carry):
            for j in range(4):
                c = 4 * g + 4 + j
                step(c, j, (j + 1) % 4)
            return carry

        nloop = (steps - 4) // 4
        lax.fori_loop(0, nloop, body, 0)
        for c in range(4 + 4 * nloop, steps):
            step(c, c % 4, (c - 3) % 4)
        for c in (steps - 3, steps - 2, steps - 1):
            b = c % 4
            wait_gather(b)
            issue_store(c, b)
        for c in range(steps - 4, steps):
            wait_store(c % 4)

    return gather_kernel(table, idx)


def _sc_scatter(msg, idx, zeros, n):
    e, d = msg.shape
    per_core = e // _NC
    per_w = per_core // _NS
    chunk = 80 if per_w % 80 == 0 else 40
    steps = per_w // chunk
    rpt = (-(-n // _NS) + 7) & ~7
    mesh = plsc.VectorSubcoreMesh(core_axis_name="c", subcore_axis_name="s")

    @functools.partial(
        pl.kernel,
        out_type=jax.ShapeDtypeStruct((_NC * n, d), jnp.float32),
        mesh=mesh,
        scratch_types=[
            [pltpu.VMEM((chunk,), jnp.int32) for _ in range(4)],
            [pltpu.VMEM((chunk, d), jnp.float32) for _ in range(4)],
            pltpu.VMEM_SHARED((n, d), jnp.float32),
            [pltpu.SemaphoreType.DMA for _ in range(4)],
            [pltpu.SemaphoreType.DMA for _ in range(4)],
            [pltpu.SemaphoreType.DMA for _ in range(4)],
        ],
    )
    def scatter_kernel(msg_hbm, idx_hbm, zeros_hbm, out_hbm,
                       idxb, msgb, acc, si, sm, sa):
        cc = lax.axis_index("c")
        s = lax.axis_index("s")
        rbase = pl.multiple_of(jnp.minimum(s * rpt, n - rpt), 8)
        pltpu.sync_copy(zeros_hbm, acc.at[pl.ds(rbase, rpt)])
        plsc.subcore_barrier()

        ebase = cc * per_core + s * per_w

        def issue_loads(c, b):
            off = ebase + c * chunk
            pltpu.async_copy(idx_hbm.at[pl.ds(off, chunk)], idxb[b], si[b])
            pltpu.async_copy(msg_hbm.at[pl.ds(off, chunk)], msgb[b], sm[b])

        def wait_loads(b):
            pltpu.make_async_copy(idx_hbm.at[pl.ds(0, chunk)], idxb[b],
                                  si[b]).wait()
            pltpu.make_async_copy(msg_hbm.at[pl.ds(0, chunk)], msgb[b],
                                  sm[b]).wait()

        def issue_add(b):
            pltpu.async_copy(msgb[b], acc.at[idxb[b]], sa[b], add=True)

        def wait_add(b):
            pltpu.make_async_copy(msgb[b], acc.at[pl.ds(0, chunk)],
                                  sa[b]).wait()

        def step(c, b, bp, prime=False):
            if not prime:
                wait_add(b)
            issue_loads(c, b)
            wait_loads(bp)
            issue_add(bp)

        issue_loads(0, 0)
        issue_loads(1, 1)
        step(2, 2, 0, prime=True)
        step(3, 3, 1, prime=True)
        step(4, 0, 2)
        step(5, 1, 3)

        def body(g, carry):
            for j in range(4):
                c = 4 * g + 6 + j
                step(c, (2 + j) % 4, j)
            return carry

        nloop = (steps - 6) // 4
        lax.fori_loop(0, nloop, body, 0)
        for c in range(6 + 4 * nloop, steps):
            step(c, c % 4, (c - 2) % 4)
        for c in (steps - 2, steps - 1):
            b = c % 4
            wait_loads(b)
            issue_add(b)
        for c in range(steps - 4, steps):
            wait_add(c % 4)

        plsc.subcore_barrier()
        pltpu.sync_copy(acc.at[pl.ds(rbase, rpt)],
                        out_hbm.at[pl.ds(cc * n + rbase, rpt)])

    return scatter_kernel(msg, idx, zeros)


def _dense_body(g_ref, e_ref, w_ref, b_ref, gg_ref, bb_ref, o_ref):
    d = e_ref.shape[1]
    g = g_ref[...].astype(jnp.bfloat16)
    ee = e_ref[...].astype(jnp.bfloat16)
    w = w_ref[...].astype(jnp.bfloat16)
    dn = (((1,), (1,)), ((), ()))
    c1 = lax.dot_general(g, w[:, :d], dn,
                         preferred_element_type=jnp.float32)
    c1 += lax.dot_general(ee, w[:, d:], dn,
                          preferred_element_type=jnp.float32)
    c1 += b_ref[...]
    mu = jnp.mean(c1, axis=-1, keepdims=True)
    var = jnp.mean((c1 - mu) ** 2, axis=-1, keepdims=True)
    c1 = (c1 - mu) / jnp.sqrt(var + 1e-5) * gg_ref[...] + bb_ref[...]
    o_ref[...] = jax.nn.sigmoid(c1[:, :d]) * jnp.tanh(c1[:, d:])


def _dense(gathered, edge, w1, b1, g_c1, be_c1):
    e, d = edge.shape
    be = 3200
    grid = e // be
    d2 = 2 * d
    return pl.pallas_call(
        _dense_body,
        grid=(grid,),
        in_specs=[
            pl.BlockSpec((be, d), lambda i: (i, 0)),
            pl.BlockSpec((be, d), lambda i: (i, 0)),
            pl.BlockSpec((d2, d2), lambda i: (0, 0)),
            pl.BlockSpec((1, d2), lambda i: (0, 0)),
            pl.BlockSpec((1, d2), lambda i: (0, 0)),
            pl.BlockSpec((1, d2), lambda i: (0, 0)),
        ],
        out_specs=pl.BlockSpec((be, d), lambda i: (i, 0)),
        out_shape=jax.ShapeDtypeStruct((e, d), jnp.float32),
    )(gathered, edge, w1, b1.reshape(1, d2), g_c1.reshape(1, d2),
      be_c1.reshape(1, d2))


def _final_body(n_ref, a0_ref, a1_ref, g_ref, b_ref, o_ref):
    agg = a0_ref[...] + a1_ref[...]
    mu = jnp.mean(agg, axis=-1, keepdims=True)
    var = jnp.mean((agg - mu) ** 2, axis=-1, keepdims=True)
    ln = (agg - mu) / jnp.sqrt(var + 1e-5) * g_ref[...] + b_ref[...]
    o_ref[...] = jnp.tanh(n_ref[...] + ln)


def _final(node, aggs, g_bn, be_bn):
    n, d = node.shape
    bn = 1000
    blk = pl.BlockSpec((bn, d), lambda i: (i, 0))
    vec = pl.BlockSpec((1, d), lambda i: (0, 0))
    return pl.pallas_call(
        _final_body,
        grid=(n // bn,),
        in_specs=[blk] * 3 + [vec] * 2,
        out_specs=blk,
        out_shape=jax.ShapeDtypeStruct((n, d), jnp.float32),
    )(node, *aggs, g_bn.reshape(1, d), be_bn.reshape(1, d))


def kernel(node_embedding, edge_embedding, i, W1, b1, g_c1, be_c1, g_bn, be_bn):
    n, d = node_embedding.shape
    e = i.shape[0]
    idx = i.astype(jnp.int32)
    rpt = (-(-n // _NS) + 7) & ~7
    zeros = jnp.zeros((rpt, d), jnp.float32)
    gathered = _sc_gather(node_embedding, idx)
    msg = _dense(gathered, edge_embedding, W1, b1, g_c1, be_c1)
    agg2 = _sc_scatter(msg, idx, zeros, n)
    return _final(node_embedding, (agg2[:n], agg2[n:]), g_bn, be_bn)


# speedup vs baseline: 1.2478x; 1.0555x over previous
import functools

import jax
import jax.numpy as jnp
from jax import lax
from jax.experimental import pallas as pl
from jax.experimental.pallas import tpu as pltpu
from jax.experimental.pallas import tpu_sc as plsc

_NC = 2
_NS = 16


def _sc_gather(table, idx):
    n, d = table.shape
    e = idx.shape[0]
    nw = _NC * _NS
    per_w = e // nw
    dt = table.dtype
    chunk = 400 if d <= 64 else 200
    steps = per_w // chunk
    mesh = plsc.VectorSubcoreMesh(core_axis_name="c", subcore_axis_name="s")

    @functools.partial(
        pl.kernel,
        out_type=jax.ShapeDtypeStruct((e, d), dt),
        mesh=mesh,
        scratch_types=[
            pltpu.VMEM((per_w,), jnp.int32),
            [pltpu.VMEM((chunk, d), dt) for _ in range(4)],
            [pltpu.SemaphoreType.DMA for _ in range(4)],
            [pltpu.SemaphoreType.DMA for _ in range(4)],
        ],
    )
    def gather_kernel(table_hbm, idx_hbm, out_hbm, idx_f, rows, sg, ss):
        wid = lax.axis_index("s") * _NC + lax.axis_index("c")
        base = wid * per_w
        pltpu.sync_copy(idx_hbm.at[pl.ds(base, per_w)], idx_f)

        def issue_gather(c, b):
            pltpu.async_copy(
                table_hbm.at[idx_f.at[pl.ds(c * chunk, chunk)]], rows[b], sg[b])

        def wait_gather(b):
            pltpu.make_async_copy(
                out_hbm.at[pl.ds(0, chunk)], rows[b], sg[b]).wait()

        def issue_store(c, b):
            pltpu.async_copy(rows[b], out_hbm.at[pl.ds(base + c * chunk, chunk)],
                             ss[b])

        def wait_store(b):
            pltpu.make_async_copy(
                rows[b], out_hbm.at[pl.ds(0, chunk)], ss[b]).wait()

        def step(c, b, bp, prime=False):
            if not prime:
                wait_store(b)
            issue_gather(c, b)
            wait_gather(bp)
            issue_store(c - 3, bp)

        issue_gather(0, 0)
        issue_gather(1, 1)
        issue_gather(2, 2)
        step(3, 3, 0, prime=True)

        def body(g, carry):
            for j in range(4):
                c = 4 * g + 4 + j
                step(c, j, (j + 1) % 4)
            return carry

        nloop = (steps - 4) // 4
        lax.fori_loop(0, nloop, body, 0)
        for c in range(4 + 4 * nloop, steps):
            step(c, c % 4, (c - 3) % 4)
        for c in (steps - 3, steps - 2, steps - 1):
            b = c % 4
            wait_gather(b)
            issue_store(c, b)
        for c in range(steps - 4, steps):
            wait_store(c % 4)

    return gather_kernel(table, idx)


def _sc_scatter(msg, idx, zeros, n):
    e, d = msg.shape
    per_core = e // _NC
    per_w = per_core // _NS
    chunk = 80 if per_w % 80 == 0 else 40
    steps = per_w // chunk
    rpt = (-(-n // _NS) + 7) & ~7
    mesh = plsc.VectorSubcoreMesh(core_axis_name="c", subcore_axis_name="s")

    @functools.partial(
        pl.kernel,
        out_type=jax.ShapeDtypeStruct((_NC * n, d), jnp.float32),
        mesh=mesh,
        scratch_types=[
            [pltpu.VMEM((chunk,), jnp.int32) for _ in range(4)],
            [pltpu.VMEM((chunk, d), jnp.float32) for _ in range(4)],
            pltpu.VMEM_SHARED((n, d), jnp.float32),
            [pltpu.SemaphoreType.DMA for _ in range(4)],
            [pltpu.SemaphoreType.DMA for _ in range(4)],
            [pltpu.SemaphoreType.DMA for _ in range(4)],
        ],
    )
    def scatter_kernel(msg_hbm, idx_hbm, zeros_hbm, out_hbm,
                       idxb, msgb, acc, si, sm, sa):
        cc = lax.axis_index("c")
        s = lax.axis_index("s")
        rbase = pl.multiple_of(jnp.minimum(s * rpt, n - rpt), 8)
        pltpu.sync_copy(zeros_hbm, acc.at[pl.ds(rbase, rpt)])
        plsc.subcore_barrier()

        ebase = cc * per_core + s * per_w

        def issue_loads(c, b):
            off = ebase + c * chunk
            pltpu.async_copy(idx_hbm.at[pl.ds(off, chunk)], idxb[b], si[b])
            pltpu.async_copy(msg_hbm.at[pl.ds(off, chunk)], msgb[b], sm[b])

        def wait_loads(b):
            pltpu.make_async_copy(idx_hbm.at[pl.ds(0, chunk)], idxb[b],
                                  si[b]).wait()
            pltpu.make_async_copy(msg_hbm.at[pl.ds(0, chunk)], msgb[b],
                                  sm[b]).wait()

        def issue_add(b):
            pltpu.async_copy(msgb[b], acc.at[idxb[b]], sa[b], add=True)

        def wait_add(b):
            pltpu.make_async_copy(msgb[b], acc.at[pl.ds(0, chunk)],
                                  sa[b]).wait()

        def step(c, b, bp, prime=False):
            if not prime:
                wait_add(b)
            issue_loads(c, b)
            wait_loads(bp)
            issue_add(bp)

        issue_loads(0, 0)
        issue_loads(1, 1)
        step(2, 2, 0, prime=True)
        step(3, 3, 1, prime=True)
        step(4, 0, 2)
        step(5, 1, 3)

        def body(g, carry):
            for j in range(4):
                c = 4 * g + 6 + j
                step(c, (2 + j) % 4, j)
            return carry

        nloop = (steps - 6) // 4
        lax.fori_loop(0, nloop, body, 0)
        for c in range(6 + 4 * nloop, steps):
            step(c, c % 4, (c - 2) % 4)
        for c in (steps - 2, steps - 1):
            b = c % 4
            wait_loads(b)
            issue_add(b)
        for c in range(steps - 4, steps):
            wait_add(c % 4)

        plsc.subcore_barrier()
        pltpu.sync_copy(acc.at[pl.ds(rbase, rpt)],
                        out_hbm.at[pl.ds(cc * n + rbase, rpt)])

    return scatter_kernel(msg, idx, zeros)


def _dense_body(g_ref, e_ref, w_ref, b_ref, gg_ref, bb_ref, o_ref):
    d = e_ref.shape[1]
    g = g_ref[...].astype(jnp.bfloat16)
    ee = e_ref[...].astype(jnp.bfloat16)
    w = w_ref[...].astype(jnp.bfloat16)
    dn = (((1,), (1,)), ((), ()))
    c1 = lax.dot_general(g, w[:, :d], dn,
                         preferred_element_type=jnp.float32)
    c1 += lax.dot_general(ee, w[:, d:], dn,
                          preferred_element_type=jnp.float32)
    c1 += b_ref[...]
    mu = jnp.mean(c1, axis=-1, keepdims=True)
    var = jnp.mean((c1 - mu) ** 2, axis=-1, keepdims=True)
    c1 = (c1 - mu) / jnp.sqrt(var + 1e-5) * gg_ref[...] + bb_ref[...]
    o_ref[...] = jax.nn.sigmoid(c1[:, :d]) * jnp.tanh(c1[:, d:])


def _dense(gathered, edge, w1, b1, g_c1, be_c1):
    e, d = edge.shape
    be = 6400
    grid = e // be
    d2 = 2 * d
    return pl.pallas_call(
        _dense_body,
        grid=(grid,),
        in_specs=[
            pl.BlockSpec((be, d), lambda i: (i, 0)),
            pl.BlockSpec((be, d), lambda i: (i, 0)),
            pl.BlockSpec((d2, d2), lambda i: (0, 0)),
            pl.BlockSpec((1, d2), lambda i: (0, 0)),
            pl.BlockSpec((1, d2), lambda i: (0, 0)),
            pl.BlockSpec((1, d2), lambda i: (0, 0)),
        ],
        out_specs=pl.BlockSpec((be, d), lambda i: (i, 0)),
        out_shape=jax.ShapeDtypeStruct((e, d), jnp.float32),
    )(gathered, edge, w1, b1.reshape(1, d2), g_c1.reshape(1, d2),
      be_c1.reshape(1, d2))


def _final_body(n_ref, a0_ref, a1_ref, g_ref, b_ref, o_ref):
    agg = a0_ref[...] + a1_ref[...]
    mu = jnp.mean(agg, axis=-1, keepdims=True)
    var = jnp.mean((agg - mu) ** 2, axis=-1, keepdims=True)
    ln = (agg - mu) / jnp.sqrt(var + 1e-5) * g_ref[...] + b_ref[...]
    o_ref[...] = jnp.tanh(n_ref[...] + ln)


def _final(node, aggs, g_bn, be_bn):
    n, d = node.shape
    bn = 1000
    blk = pl.BlockSpec((bn, d), lambda i: (i, 0))
    vec = pl.BlockSpec((1, d), lambda i: (0, 0))
    return pl.pallas_call(
        _final_body,
        grid=(n // bn,),
        in_specs=[blk] * 3 + [vec] * 2,
        out_specs=blk,
        out_shape=jax.ShapeDtypeStruct((n, d), jnp.float32),
    )(node, *aggs, g_bn.reshape(1, d), be_bn.reshape(1, d))


def kernel(node_embedding, edge_embedding, i, W1, b1, g_c1, be_c1, g_bn, be_bn):
    n, d = node_embedding.shape
    e = i.shape[0]
    idx = i.astype(jnp.int32)
    rpt = (-(-n // _NS) + 7) & ~7
    zeros = jnp.zeros((rpt, d), jnp.float32)
    gathered = _sc_gather(node_embedding, idx)
    msg = _dense(gathered, edge_embedding, W1, b1, g_c1, be_c1)
    agg2 = _sc_scatter(msg, idx, zeros, n)
    return _final(node_embedding, (agg2[:n], agg2[n:]), g_bn, be_bn)


# speedup vs baseline: 1.2722x; 1.0195x over previous
import functools

import jax
import jax.numpy as jnp
from jax import lax
from jax.experimental import pallas as pl
from jax.experimental.pallas import tpu as pltpu
from jax.experimental.pallas import tpu_sc as plsc

_NC = 2
_NS = 16


def _sc_gather(table, idx):
    n, d = table.shape
    e = idx.shape[0]
    nw = _NC * _NS
    per_w = e // nw
    dt = table.dtype
    chunk = 400 if d <= 64 else 200
    steps = per_w // chunk
    mesh = plsc.VectorSubcoreMesh(core_axis_name="c", subcore_axis_name="s")

    @functools.partial(
        pl.kernel,
        out_type=jax.ShapeDtypeStruct((e, d), dt),
        mesh=mesh,
        scratch_types=[
            pltpu.VMEM((per_w,), jnp.int32),
            [pltpu.VMEM((chunk, d), dt) for _ in range(4)],
            [pltpu.SemaphoreType.DMA for _ in range(4)],
            [pltpu.SemaphoreType.DMA for _ in range(4)],
        ],
    )
    def gather_kernel(table_hbm, idx_hbm, out_hbm, idx_f, rows, sg, ss):
        wid = lax.axis_index("s") * _NC + lax.axis_index("c")
        base = wid * per_w
        pltpu.sync_copy(idx_hbm.at[pl.ds(base, per_w)], idx_f)

        def issue_gather(c, b):
            pltpu.async_copy(
                table_hbm.at[idx_f.at[pl.ds(c * chunk, chunk)]], rows[b], sg[b])

        def wait_gather(b):
            pltpu.make_async_copy(
                out_hbm.at[pl.ds(0, chunk)], rows[b], sg[b]).wait()

        def issue_store(c, b):
            pltpu.async_copy(rows[b], out_hbm.at[pl.ds(base + c * chunk, chunk)],
                             ss[b])

        def wait_store(b):
            pltpu.make_async_copy(
                rows[b], out_hbm.at[pl.ds(0, chunk)], ss[b]).wait()

        def step(c, b, bp, prime=False):
            if not prime:
                wait_store(b)
            issue_gather(c, b)
            wait_gather(bp)
            issue_store(c - 3, bp)

        issue_gather(0, 0)
        issue_gather(1, 1)
        issue_gather(2, 2)
        step(3, 3, 0, prime=True)

        def body(g, carry):
            for j in range(4):
                c = 4 * g + 4 + j
                step(c, j, (j + 1) % 4)
            return carry

        nloop = (steps - 4) // 4
        lax.fori_loop(0, nloop, body, 0)
        for c in range(4 + 4 * nloop, steps):
            step(c, c % 4, (c - 3) % 4)
        for c in (steps - 3, steps - 2, steps - 1):
            b = c % 4
            wait_gather(b)
            issue_store(c, b)
        for c in range(steps - 4, steps):
            wait_store(c % 4)

    return gather_kernel(table, idx)


def _sc_scatter(msg, idx, zeros, n):
    e, d = msg.shape
    per_core = e // _NC
    per_w = per_core // _NS
    chunk = 80 if per_w % 80 == 0 else 40
    steps = per_w // chunk
    rpt = (-(-n // _NS) + 7) & ~7
    mesh = plsc.VectorSubcoreMesh(core_axis_name="c", subcore_axis_name="s")

    @functools.partial(
        pl.kernel,
        out_type=jax.ShapeDtypeStruct((_NC * n, d), jnp.float32),
        mesh=mesh,
        scratch_types=[
            [pltpu.VMEM((chunk,), jnp.int32) for _ in range(4)],
            [pltpu.VMEM((chunk, d), jnp.float32) for _ in range(4)],
            pltpu.VMEM_SHARED((n, d), jnp.float32),
            [pltpu.SemaphoreType.DMA for _ in range(4)],
            [pltpu.SemaphoreType.DMA for _ in range(4)],
            [pltpu.SemaphoreType.DMA for _ in range(4)],
        ],
    )
    def scatter_kernel(msg_hbm, idx_hbm, zeros_hbm, out_hbm,
                       idxb, msgb, acc, si, sm, sa):
        cc = lax.axis_index("c")
        s = lax.axis_index("s")
        rbase = pl.multiple_of(jnp.minimum(s * rpt, n - rpt), 8)
        pltpu.sync_copy(zeros_hbm, acc.at[pl.ds(rbase, rpt)])
        plsc.subcore_barrier()

        ebase = cc * per_core + s * per_w

        def issue_loads(c, b):
            off = ebase + c * chunk
            pltpu.async_copy(idx_hbm.at[pl.ds(off, chunk)], idxb[b], si[b])
            pltpu.async_copy(msg_hbm.at[pl.ds(off, chunk)], msgb[b], sm[b])

        def wait_loads(b):
            pltpu.make_async_copy(idx_hbm.at[pl.ds(0, chunk)], idxb[b],
                                  si[b]).wait()
            pltpu.make_async_copy(msg_hbm.at[pl.ds(0, chunk)], msgb[b],
                                  sm[b]).wait()

        def issue_add(b):
            pltpu.async_copy(msgb[b], acc.at[idxb[b]], sa[b], add=True)

        def wait_add(b):
            pltpu.make_async_copy(msgb[b], acc.at[pl.ds(0, chunk)],
                                  sa[b]).wait()

        def step(c, b, bp, prime=False):
            if not prime:
                wait_add(b)
            issue_loads(c, b)
            wait_loads(bp)
            issue_add(bp)

        issue_loads(0, 0)
        issue_loads(1, 1)
        step(2, 2, 0, prime=True)
        step(3, 3, 1, prime=True)
        step(4, 0, 2)
        step(5, 1, 3)

        def body(g, carry):
            for j in range(4):
                c = 4 * g + 6 + j
                step(c, (2 + j) % 4, j)
            return carry

        nloop = (steps - 6) // 4
        lax.fori_loop(0, nloop, body, 0)
        for c in range(6 + 4 * nloop, steps):
            step(c, c % 4, (c - 2) % 4)
        for c in (steps - 2, steps - 1):
            b = c % 4
            wait_loads(b)
            issue_add(b)
        for c in range(steps - 4, steps):
            wait_add(c % 4)

        plsc.subcore_barrier()
        pltpu.sync_copy(acc.at[pl.ds(rbase, rpt)],
                        out_hbm.at[pl.ds(cc * n + rbase, rpt)])

    return scatter_kernel(msg, idx, zeros)


def _dense_body(g_ref, e_ref, w_ref, b_ref, gg_ref, bb_ref, o_ref):
    d = e_ref.shape[1]
    g = g_ref[...].astype(jnp.bfloat16)
    ee = e_ref[...].astype(jnp.bfloat16)
    w = w_ref[...].astype(jnp.bfloat16)
    dn = (((1,), (1,)), ((), ()))
    c1 = lax.dot_general(g, w[:, :d], dn,
                         preferred_element_type=jnp.float32)
    c1 += lax.dot_general(ee, w[:, d:], dn,
                          preferred_element_type=jnp.float32)
    c1 += b_ref[...]
    mu = jnp.mean(c1, axis=-1, keepdims=True)
    var = jnp.mean((c1 - mu) ** 2, axis=-1, keepdims=True)
    c1 = (c1 - mu) / jnp.sqrt(var + 1e-5) * gg_ref[...] + bb_ref[...]
    o_ref[...] = jax.nn.sigmoid(c1[:, :d]) * jnp.tanh(c1[:, d:])


def _dense(gathered, edge, w1, b1, g_c1, be_c1):
    e, d = edge.shape
    be = 12800
    grid = e // be
    d2 = 2 * d
    return pl.pallas_call(
        _dense_body,
        grid=(grid,),
        in_specs=[
            pl.BlockSpec((be, d), lambda i: (i, 0)),
            pl.BlockSpec((be, d), lambda i: (i, 0)),
            pl.BlockSpec((d2, d2), lambda i: (0, 0)),
            pl.BlockSpec((1, d2), lambda i: (0, 0)),
            pl.BlockSpec((1, d2), lambda i: (0, 0)),
            pl.BlockSpec((1, d2), lambda i: (0, 0)),
        ],
        out_specs=pl.BlockSpec((be, d), lambda i: (i, 0)),
        out_shape=jax.ShapeDtypeStruct((e, d), jnp.float32),
    )(gathered, edge, w1, b1.reshape(1, d2), g_c1.reshape(1, d2),
      be_c1.reshape(1, d2))


def _final_body(n_ref, a0_ref, a1_ref, g_ref, b_ref, o_ref):
    agg = a0_ref[...] + a1_ref[...]
    mu = jnp.mean(agg, axis=-1, keepdims=True)
    var = jnp.mean((agg - mu) ** 2, axis=-1, keepdims=True)
    ln = (agg - mu) / jnp.sqrt(var + 1e-5) * g_ref[...] + b_ref[...]
    o_ref[...] = jnp.tanh(n_ref[...] + ln)


def _final(node, aggs, g_bn, be_bn):
    n, d = node.shape
    bn = 1000
    blk = pl.BlockSpec((bn, d), lambda i: (i, 0))
    vec = pl.BlockSpec((1, d), lambda i: (0, 0))
    return pl.pallas_call(
        _final_body,
        grid=(n // bn,),
        in_specs=[blk] * 3 + [vec] * 2,
        out_specs=blk,
        out_shape=jax.ShapeDtypeStruct((n, d), jnp.float32),
    )(node, *aggs, g_bn.reshape(1, d), be_bn.reshape(1, d))


def kernel(node_embedding, edge_embedding, i, W1, b1, g_c1, be_c1, g_bn, be_bn):
    n, d = node_embedding.shape
    e = i.shape[0]
    idx = i.astype(jnp.int32)
    rpt = (-(-n // _NS) + 7) & ~7
    zeros = jnp.zeros((rpt, d), jnp.float32)
    gathered = _sc_gather(node_embedding, idx)
    msg = _dense(gathered, edge_embedding, W1, b1, g_c1, be_c1)
    agg2 = _sc_scatter(msg, idx, zeros, n)
    return _final(node_embedding, (agg2[:n], agg2[n:]), g_bn, be_bn)


# speedup vs baseline: 1.2724x; 1.0002x over previous
import functools

import jax
import jax.numpy as jnp
from jax import lax
from jax.experimental import pallas as pl
from jax.experimental.pallas import tpu as pltpu
from jax.experimental.pallas import tpu_sc as plsc

_NC = 2
_NS = 16


def _sc_gather(table, idx):
    n, d = table.shape
    e = idx.shape[0]
    nw = _NC * _NS
    per_w = e // nw
    dt = table.dtype
    chunk = 400 if d <= 64 else 200
    steps = per_w // chunk
    mesh = plsc.VectorSubcoreMesh(core_axis_name="c", subcore_axis_name="s")

    @functools.partial(
        pl.kernel,
        out_type=jax.ShapeDtypeStruct((e, d), dt),
        mesh=mesh,
        scratch_types=[
            pltpu.VMEM((per_w,), jnp.int32),
            [pltpu.VMEM((chunk, d), dt) for _ in range(4)],
            [pltpu.SemaphoreType.DMA for _ in range(4)],
            [pltpu.SemaphoreType.DMA for _ in range(4)],
        ],
    )
    def gather_kernel(table_hbm, idx_hbm, out_hbm, idx_f, rows, sg, ss):
        wid = lax.axis_index("s") * _NC + lax.axis_index("c")
        base = wid * per_w
        pltpu.sync_copy(idx_hbm.at[pl.ds(base, per_w)], idx_f)

        def issue_gather(c, b):
            pltpu.async_copy(
                table_hbm.at[idx_f.at[pl.ds(c * chunk, chunk)]], rows[b], sg[b])

        def wait_gather(b):
            pltpu.make_async_copy(
                out_hbm.at[pl.ds(0, chunk)], rows[b], sg[b]).wait()

        def issue_store(c, b):
            pltpu.async_copy(rows[b], out_hbm.at[pl.ds(base + c * chunk, chunk)],
                             ss[b])

        def wait_store(b):
            pltpu.make_async_copy(
                rows[b], out_hbm.at[pl.ds(0, chunk)], ss[b]).wait()

        def step(c, b, bp, prime=False):
            if not prime:
                wait_store(b)
            issue_gather(c, b)
            wait_gather(bp)
            issue_store(c - 3, bp)

        issue_gather(0, 0)
        issue_gather(1, 1)
        issue_gather(2, 2)
        step(3, 3, 0, prime=True)

        def body(g, carry):
            for j in range(4):
                c = 4 * g + 4 + j
                step(c, j, (j + 1) % 4)
            return carry

        nloop = (steps - 4) // 4
        lax.fori_loop(0, nloop, body, 0)
        for c in range(4 + 4 * nloop, steps):
            step(c, c % 4, (c - 3) % 4)
        for c in (steps - 3, steps - 2, steps - 1):
            b = c % 4
            wait_gather(b)
            issue_store(c, b)
        for c in range(steps - 4, steps):
            wait_store(c % 4)

    return gather_kernel(table, idx)


def _sc_scatter(msg, idx, zeros, n):
    e, d = msg.shape
    per_core = e // _NC
    per_w = per_core // _NS
    chunk = 80 if per_w % 80 == 0 else 40
    steps = per_w // chunk
    rpt = (-(-n // _NS) + 7) & ~7
    mesh = plsc.VectorSubcoreMesh(core_axis_name="c", subcore_axis_name="s")

    @functools.partial(
        pl.kernel,
        out_type=jax.ShapeDtypeStruct((_NC * n, d), jnp.float32),
        mesh=mesh,
        scratch_types=[
            [pltpu.VMEM((chunk,), jnp.int32) for _ in range(4)],
            [pltpu.VMEM((chunk, d), jnp.float32) for _ in range(4)],
            pltpu.VMEM_SHARED((n, d), jnp.float32),
            [pltpu.SemaphoreType.DMA for _ in range(4)],
            [pltpu.SemaphoreType.DMA for _ in range(4)],
            [pltpu.SemaphoreType.DMA for _ in range(4)],
        ],
    )
    def scatter_kernel(msg_hbm, idx_hbm, zeros_hbm, out_hbm,
                       idxb, msgb, acc, si, sm, sa):
        cc = lax.axis_index("c")
        s = lax.axis_index("s")
        rbase = pl.multiple_of(jnp.minimum(s * rpt, n - rpt), 8)
        pltpu.sync_copy(zeros_hbm, acc.at[pl.ds(rbase, rpt)])
        plsc.subcore_barrier()

        ebase = cc * per_core + s * per_w

        def issue_loads(c, b):
            off = ebase + c * chunk
            pltpu.async_copy(idx_hbm.at[pl.ds(off, chunk)], idxb[b], si[b])
            pltpu.async_copy(msg_hbm.at[pl.ds(off, chunk)], msgb[b], sm[b])

        def wait_loads(b):
            pltpu.make_async_copy(idx_hbm.at[pl.ds(0, chunk)], idxb[b],
                                  si[b]).wait()
            pltpu.make_async_copy(msg_hbm.at[pl.ds(0, chunk)], msgb[b],
                                  sm[b]).wait()

        def issue_add(b):
            pltpu.async_copy(msgb[b], acc.at[idxb[b]], sa[b], add=True)

        def wait_add(b):
            pltpu.make_async_copy(msgb[b], acc.at[pl.ds(0, chunk)],
                                  sa[b]).wait()

        def step(c, b, bp, prime=False):
            if not prime:
                wait_add(b)
            issue_loads(c, b)
            wait_loads(bp)
            issue_add(bp)

        issue_loads(0, 0)
        issue_loads(1, 1)
        step(2, 2, 0, prime=True)
        step(3, 3, 1, prime=True)
        step(4, 0, 2)
        step(5, 1, 3)

        def body(g, carry):
            for j in range(4):
                c = 4 * g + 6 + j
                step(c, (2 + j) % 4, j)
            return carry

        nloop = (steps - 6) // 4
        lax.fori_loop(0, nloop, body, 0)
        for c in range(6 + 4 * nloop, steps):
            step(c, c % 4, (c - 2) % 4)
        for c in (steps - 2, steps - 1):
            b = c % 4
            wait_loads(b)
            issue_add(b)
        for c in range(steps - 4, steps):
            wait_add(c % 4)

        plsc.subcore_barrier()
        pltpu.sync_copy(acc.at[pl.ds(rbase, rpt)],
                        out_hbm.at[pl.ds(cc * n + rbase, rpt)])

    return scatter_kernel(msg, idx, zeros)


def _dense_body(g_ref, e_ref, w_ref, b_ref, gg_ref, bb_ref, o_ref):
    d = e_ref.shape[1]
    g = g_ref[...].astype(jnp.bfloat16)
    ee = e_ref[...].astype(jnp.bfloat16)
    w = w_ref[...].astype(jnp.bfloat16)
    dn = (((1,), (1,)), ((), ()))
    c1 = lax.dot_general(g, w[:, :d], dn,
                         preferred_element_type=jnp.float32)
    c1 += lax.dot_general(ee, w[:, d:], dn,
                          preferred_element_type=jnp.float32)
    c1 += b_ref[...]
    mu = jnp.mean(c1, axis=-1, keepdims=True)
    var = jnp.mean((c1 - mu) ** 2, axis=-1, keepdims=True)
    c1 = (c1 - mu) / jnp.sqrt(var + 1e-5) * gg_ref[...] + bb_ref[...]
    o_ref[...] = jax.nn.sigmoid(c1[:, :d]) * jnp.tanh(c1[:, d:])


def _dense(gathered, edge, w1, b1, g_c1, be_c1):
    e, d = edge.shape
    be = 16000
    grid = e // be
    d2 = 2 * d
    return pl.pallas_call(
        _dense_body,
        grid=(grid,),
        in_specs=[
            pl.BlockSpec((be, d), lambda i: (i, 0)),
            pl.BlockSpec((be, d), lambda i: (i, 0)),
            pl.BlockSpec((d2, d2), lambda i: (0, 0)),
            pl.BlockSpec((1, d2), lambda i: (0, 0)),
            pl.BlockSpec((1, d2), lambda i: (0, 0)),
            pl.BlockSpec((1, d2), lambda i: (0, 0)),
        ],
        out_specs=pl.BlockSpec((be, d), lambda i: (i, 0)),
        out_shape=jax.ShapeDtypeStruct((e, d), jnp.float32),
    )(gathered, edge, w1, b1.reshape(1, d2), g_c1.reshape(1, d2),
      be_c1.reshape(1, d2))


def _final_body(n_ref, a0_ref, a1_ref, g_ref, b_ref, o_ref):
    agg = a0_ref[...] + a1_ref[...]
    mu = jnp.mean(agg, axis=-1, keepdims=True)
    var = jnp.mean((agg - mu) ** 2, axis=-1, keepdims=True)
    ln = (agg - mu) / jnp.sqrt(var + 1e-5) * g_ref[...] + b_ref[...]
    o_ref[...] = jnp.tanh(n_ref[...] + ln)


def _final(node, aggs, g_bn, be_bn):
    n, d = node.shape
    bn = 1000
    blk = pl.BlockSpec((bn, d), lambda i: (i, 0))
    vec = pl.BlockSpec((1, d), lambda i: (0, 0))
    return pl.pallas_call(
        _final_body,
        grid=(n // bn,),
        in_specs=[blk] * 3 + [vec] * 2,
        out_specs=blk,
        out_shape=jax.ShapeDtypeStruct((n, d), jnp.float32),
    )(node, *aggs, g_bn.reshape(1, d), be_bn.reshape(1, d))


def kernel(node_embedding, edge_embedding, i, W1, b1, g_c1, be_c1, g_bn, be_bn):
    n, d = node_embedding.shape
    e = i.shape[0]
    idx = i.astype(jnp.int32)
    rpt = (-(-n // _NS) + 7) & ~7
    zeros = jnp.zeros((rpt, d), jnp.float32)
    gathered = _sc_gather(node_embedding, idx)
    msg = _dense(gathered, edge_embedding, W1, b1, g_c1, be_c1)
    agg2 = _sc_scatter(msg, idx, zeros, n)
    return _final(node_embedding, (agg2[:n], agg2[n:]), g_bn, be_bn)
